# baseline (speedup 1.0000x reference)
import functools

import jax
import jax.numpy as jnp
from jax.experimental import pallas as pl
from jax.experimental.pallas import tpu as pltpu

_MARGIN = 0.3
_TILE_C = 2048

_SP_C0 = 0.6931473570802212
_SP_C1 = 0.12499416966835278
_SP_C2 = -0.005178683812392345
_SP_C3 = 0.00029877731655706833


def _softplus_poly(x):
    u = x * x
    p = (_SP_C3 * u + _SP_C2) * u + _SP_C1
    return (p * u + _SP_C0) + 0.5 * x


def _softplus(x):
    return jnp.logaddexp(x, 0.0)


def _rowsum(a):
    ones = jnp.ones((a.shape[1], 1), jnp.float32)
    return jax.lax.dot_general(a, ones, (((1,), (0,)), ((), ())),
                               preferred_element_type=jnp.float32)


def _colsum(a):
    ones = jnp.ones((1, a.shape[0]), jnp.float32)
    return jax.lax.dot_general(ones, a, (((1,), (0,)), ((), ())),
                               preferred_element_type=jnp.float32)


def _main_kernel(x_ref, v_ref, tgt_ref,
                 out_ref, s_ref, tl_ref, th_ref,
                 *, C, margin):
    j = pl.program_id(0)
    nj = pl.num_programs(0)
    Ct = v_ref.shape[0]

    x = x_ref[...]
    invn = jax.lax.rsqrt(
        jnp.maximum(jnp.sum(x * x, axis=1, keepdims=True), 1e-24))

    @pl.when(j == 0)
    def _init():
        s_ref[...] = jnp.zeros(s_ref.shape, jnp.float32)
        tl_ref[...] = jnp.zeros(tl_ref.shape, jnp.float32)
        th_ref[...] = jnp.zeros(th_ref.shape, jnp.float32)

    block = jax.lax.dot_general(x, v_ref[...], (((1,), (1,)), ((), ())),
                                preferred_element_type=jnp.float32)
    out_ref[...] = block

    cols = j * Ct + jax.lax.broadcasted_iota(jnp.int32, (1, Ct), 1)
    tmask = cols == tgt_ref[...]

    def accum(blk):
        tmask_f = tmask.astype(jnp.float32)
        s_ref[...] += _rowsum(jnp.exp(blk))
        tl_ref[...] += _rowsum(tmask_f * blk)
        tsims = blk * invn
        thpsim_raw = jnp.min(jnp.where(tmask, tsims, 1e30), axis=0,
                             keepdims=True)
        has_pos = thpsim_raw < 1e29
        thpsim = jnp.where(has_pos, thpsim_raw, 0.0)
        tthrd = jnp.where(has_pos, thpsim - margin, 1.0 - margin)
        self_f = (tsims > tthrd).astype(jnp.float32) - tmask_f
        tcnt = _colsum(self_f)
        tsum = _colsum(self_f * _softplus_poly(tsims))
        thn = jnp.where(tcnt > 0.0, tsum / jnp.maximum(tcnt, 1.0), 0.0)
        thp = jnp.where(has_pos, _softplus(-thpsim), 0.0)
        th_ref[...] += thp + thn

    @pl.when(j < nj - 1)
    def _full():
        accum(block)

    @pl.when(j == nj - 1)
    def _tail():
        accum(jnp.where(cols < C, block, 0.0))


def _finish_kernel(x_ref, pairs_ref, s_ref, tl_ref, th_ref, loss_ref,
                   *, C, P, margin, npad):
    B = x_ref.shape[0]
    x = x_ref[...]
    invn = jax.lax.rsqrt(
        jnp.maximum(jnp.sum(x * x, axis=1, keepdims=True), 1e-24))

    bu = jnp.mean(jnp.log(s_ref[...] - float(npad)) - tl_ref[...])
    th_loss = jnp.sum(th_ref[...]) / C

    ninp = x * invn
    sim = jax.lax.dot_general(ninp, ninp, (((1,), (1,)), ((), ())),
                              preferred_element_type=jnp.float32)
    colid = jax.lax.broadcasted_iota(jnp.int32, (1, B), 1)
    pairs = pairs_ref[...]
    hp = jnp.full((B, 1), 2.0, jnp.float32)
    for q in range(P):
        pid = pairs[:, q:q + 1]
        ps = jnp.sum(jnp.where(colid == pid, sim, 0.0), axis=1, keepdims=True)
        hp = jnp.minimum(hp, ps)
    thr = hp - margin
    cnt = jnp.zeros((B, 1), jnp.float32)
    nsum = jnp.zeros((B, 1), jnp.float32)
    for q in range(P):
        nid = pairs[:, P + q:P + q + 1]
        ns = jnp.sum(jnp.where(colid == nid, sim, 0.0), axis=1, keepdims=True)
        sel = ns > thr
        cnt += sel.astype(jnp.float32)
        nsum += jnp.where(sel, _softplus(ns), 0.0)
    hn = jnp.where(cnt > 0.0, nsum / jnp.maximum(cnt, 1.0), 0.0)
    h_loss = jnp.mean(_softplus(-hp) + hn)

    loss_ref[...] = jnp.full(loss_ref.shape, bu + h_loss + th_loss,
                             jnp.float32)


def _run(inputs, V, tgt2, pairs, tile_c, interpret=False):
    B, D = inputs.shape
    C = V.shape[0]
    P = pairs.shape[1] // 2
    grid = pl.cdiv(C, tile_c)
    npad = grid * tile_c - C

    main = functools.partial(_main_kernel, C=C, margin=_MARGIN)
    out, s, tl, th = pl.pallas_call(
        main,
        grid=(grid,),
        in_specs=[
            pl.BlockSpec((B, D), lambda j: (0, 0)),
            pl.BlockSpec((tile_c, D), lambda j: (j, 0)),
            pl.BlockSpec((B, 1), lambda j: (0, 0)),
        ],
        out_specs=[
            pl.BlockSpec((B, tile_c), lambda j: (0, j)),
            pl.BlockSpec((B, 1), lambda j: (0, 0)),
            pl.BlockSpec((B, 1), lambda j: (0, 0)),
            pl.BlockSpec((1, tile_c), lambda j: (0, 0)),
        ],
        out_shape=[
            jax.ShapeDtypeStruct((B, C), jnp.float32),
            jax.ShapeDtypeStruct((B, 1), jnp.float32),
            jax.ShapeDtypeStruct((B, 1), jnp.float32),
            jax.ShapeDtypeStruct((1, tile_c), jnp.float32),
        ],
        compiler_params=pltpu.CompilerParams(
            dimension_semantics=("arbitrary",)),
        interpret=interpret,
    )(inputs, V, tgt2)

    finish = functools.partial(_finish_kernel, C=C, P=P, margin=_MARGIN,
                               npad=npad)
    loss = pl.pallas_call(
        finish,
        out_shape=jax.ShapeDtypeStruct((8, 128), jnp.float32),
        interpret=interpret,
    )(inputs, pairs, s, tl, th)
    return out, loss


def kernel(inputs, V, targets, label_to_pairs, indexs):
    B, D = inputs.shape
    P = label_to_pairs.shape[2]
    tgt2 = targets.astype(jnp.int32).reshape(B, 1)
    pairs = label_to_pairs.astype(jnp.int32).reshape(B, 2 * P)
    out, loss = _run(inputs, V, tgt2, pairs, _TILE_C)
    return loss[0, 0], out


# speedup vs baseline: 1.0823x; 1.0823x over previous
import functools

import jax
import jax.numpy as jnp
from jax.experimental import pallas as pl
from jax.experimental.pallas import tpu as pltpu

_MARGIN = 0.3
_TILE_C = 2048

_SP_C0 = 0.6931560237943111
_SP_C1 = 0.12482909105071764
_SP_C2 = -0.00472949478678556


def _softplus_poly(x):
    u = x * x
    return ((_SP_C2 * u + _SP_C1) * u + _SP_C0) + 0.5 * x


def _softplus(x):
    return jnp.logaddexp(x, 0.0)


def _main_kernel(x_ref, v_ref, tgt_ref,
                 out_ref, s_ref, tl_ref, th_ref,
                 *, C, margin):
    j = pl.program_id(0)
    nj = pl.num_programs(0)
    Ct = v_ref.shape[0]

    x = x_ref[...]
    invn = jax.lax.rsqrt(
        jnp.maximum(jnp.sum(x * x, axis=1, keepdims=True), 1e-24))

    @pl.when(j == 0)
    def _init():
        s_ref[...] = jnp.zeros(s_ref.shape, jnp.float32)
        tl_ref[...] = jnp.zeros(tl_ref.shape, jnp.float32)
        th_ref[...] = jnp.zeros(th_ref.shape, jnp.float32)

    block = jax.lax.dot_general(x, v_ref[...], (((1,), (1,)), ((), ())),
                                preferred_element_type=jnp.float32)
    out_ref[...] = block

    cols = j * Ct + jax.lax.broadcasted_iota(jnp.int32, (1, Ct), 1)
    tmask = cols == tgt_ref[...]

    def accum(blk):
        tmask_f = tmask.astype(jnp.float32)
        s_ref[...] += jnp.sum(jnp.exp(blk), axis=1, keepdims=True)
        tsims = blk * invn
        tl_ref[...] += jnp.sum(tmask_f * tsims, axis=1, keepdims=True)
        thpsim_raw = jnp.min(jnp.where(tmask, tsims, 1e30), axis=0,
                             keepdims=True)
        has_pos = thpsim_raw < 1e29
        thpsim = jnp.where(has_pos, thpsim_raw, 0.0)
        tthrd = jnp.where(has_pos, thpsim - margin, 1.0 - margin)
        self_f = (tsims > tthrd).astype(jnp.float32) - tmask_f
        tcnt = jnp.sum(self_f, axis=0, keepdims=True)
        tsum = jnp.sum(self_f * _softplus_poly(tsims), axis=0, keepdims=True)
        thn = jnp.where(tcnt > 0.0, tsum / jnp.maximum(tcnt, 1.0), 0.0)
        thp = jnp.where(has_pos, _softplus(-thpsim), 0.0)
        th_ref[...] += thp + thn

    accum(jnp.where(cols < C, block, 0.0))


def _finish_kernel(x_ref, pairs_ref, s_ref, tl_ref, th_ref, loss_ref,
                   *, C, P, margin, npad):
    B = x_ref.shape[0]
    x = x_ref[...]
    invn = jax.lax.rsqrt(
        jnp.maximum(jnp.sum(x * x, axis=1, keepdims=True), 1e-24))

    bu = jnp.mean(jnp.log(s_ref[...] - float(npad)) - tl_ref[...] / invn)
    th_loss = jnp.sum(th_ref[...]) / C

    ninp = x * invn
    sim = jax.lax.dot_general(ninp, ninp, (((1,), (1,)), ((), ())),
                              preferred_element_type=jnp.float32)
    colid = jax.lax.broadcasted_iota(jnp.int32, (1, B), 1)
    pairs = pairs_ref[...]
    hp = jnp.full((B, 1), 2.0, jnp.float32)
    for q in range(P):
        pid = pairs[:, q:q + 1]
        ps = jnp.sum(jnp.where(colid == pid, sim, 0.0), axis=1, keepdims=True)
        hp = jnp.minimum(hp, ps)
    thr = hp - margin
    cnt = jnp.zeros((B, 1), jnp.float32)
    nsum = jnp.zeros((B, 1), jnp.float32)
    for q in range(P):
        nid = pairs[:, P + q:P + q + 1]
        ns = jnp.sum(jnp.where(colid == nid, sim, 0.0), axis=1, keepdims=True)
        sel = ns > thr
        cnt += sel.astype(jnp.float32)
        nsum += jnp.where(sel, _softplus(ns), 0.0)
    hn = jnp.where(cnt > 0.0, nsum / jnp.maximum(cnt, 1.0), 0.0)
    h_loss = jnp.mean(_softplus(-hp) + hn)

    loss_ref[...] = jnp.full(loss_ref.shape, bu + h_loss + th_loss,
                             jnp.float32)


def _run(inputs, V, tgt2, pairs, tile_c, interpret=False):
    B, D = inputs.shape
    C = V.shape[0]
    P = pairs.shape[1] // 2
    grid = pl.cdiv(C, tile_c)
    npad = grid * tile_c - C

    main = functools.partial(_main_kernel, C=C, margin=_MARGIN)
    out, s, tl, th = pl.pallas_call(
        main,
        grid=(grid,),
        in_specs=[
            pl.BlockSpec((B, D), lambda j: (0, 0)),
            pl.BlockSpec((tile_c, D), lambda j: (j, 0)),
            pl.BlockSpec((B, 1), lambda j: (0, 0)),
        ],
        out_specs=[
            pl.BlockSpec((B, tile_c), lambda j: (0, j)),
            pl.BlockSpec((B, 1), lambda j: (0, 0)),
            pl.BlockSpec((B, 1), lambda j: (0, 0)),
            pl.BlockSpec((1, tile_c), lambda j: (0, 0)),
        ],
        out_shape=[
            jax.ShapeDtypeStruct((B, C), jnp.float32),
            jax.ShapeDtypeStruct((B, 1), jnp.float32),
            jax.ShapeDtypeStruct((B, 1), jnp.float32),
            jax.ShapeDtypeStruct((1, tile_c), jnp.float32),
        ],
        compiler_params=pltpu.CompilerParams(
            dimension_semantics=("arbitrary",)),
        interpret=interpret,
    )(inputs, V, tgt2)

    finish = functools.partial(_finish_kernel, C=C, P=P, margin=_MARGIN,
                               npad=npad)
    loss = pl.pallas_call(
        finish,
        out_shape=jax.ShapeDtypeStruct((8, 128), jnp.float32),
        interpret=interpret,
    )(inputs, pairs, s, tl, th)
    return out, loss


def kernel(inputs, V, targets, label_to_pairs, indexs):
    B, D = inputs.shape
    P = label_to_pairs.shape[2]
    tgt2 = targets.astype(jnp.int32).reshape(B, 1)
    pairs = label_to_pairs.astype(jnp.int32).reshape(B, 2 * P)
    out, loss = _run(inputs, V, tgt2, pairs, _TILE_C)
    return loss[0, 0], out


# speedup vs baseline: 1.9057x; 1.7608x over previous
import functools

import jax
import jax.numpy as jnp
from jax.experimental import pallas as pl
from jax.experimental.pallas import tpu as pltpu

_MARGIN = 0.3
_TILE_C = 1536

_SP_C0 = 0.6931560237943111
_SP_C1 = 0.12482909105071764
_SP_C2 = -0.00472949478678556


def _softplus_poly(x):
    u = x * x
    return ((_SP_C2 * u + _SP_C1) * u + _SP_C0) + 0.5 * x


def _softplus(x):
    return jnp.logaddexp(x, 0.0)


def _rownorms_sq(x):
    ones = jnp.ones((1, x.shape[1]), jnp.float32)
    return jax.lax.dot_general(ones, x * x, (((1,), (1,)), ((), ())),
                               preferred_element_type=jnp.float32)


def _main_kernel(x_ref, v_ref, tgt_ref,
                 out_ref, s_ref, tl_ref, th_ref, invn_ref,
                 *, C, margin):
    j = pl.program_id(0)
    Ct = v_ref.shape[0]

    x = x_ref[...]

    @pl.when(j == 0)
    def _init():
        s_ref[...] = jnp.zeros(s_ref.shape, jnp.float32)
        tl_ref[...] = jnp.zeros(tl_ref.shape, jnp.float32)
        th_ref[...] = jnp.zeros(th_ref.shape, jnp.float32)
        invn_ref[...] = jax.lax.rsqrt(
            jnp.maximum(_rownorms_sq(x), 1e-24))

    blockT = jax.lax.dot_general(v_ref[...], x, (((1,), (1,)), ((), ())),
                                 preferred_element_type=jnp.float32)
    out_ref[...] = blockT

    rows = j * Ct + jax.lax.broadcasted_iota(jnp.int32, (Ct, 1), 0)
    tmask = rows == tgt_ref[...]
    blk = jnp.where(rows < C, blockT, 0.0)

    tmask_f = tmask.astype(jnp.float32)
    s_ref[...] += jnp.sum(jnp.exp(blk), axis=0, keepdims=True)
    invn = invn_ref[...]
    tsims = blk * invn
    tl_ref[...] += jnp.sum(tmask_f * tsims, axis=0, keepdims=True)
    thpsim_raw = jnp.min(jnp.where(tmask, tsims, 1e30), axis=1, keepdims=True)
    has_pos = thpsim_raw < 1e29
    thpsim = jnp.where(has_pos, thpsim_raw, 0.0)
    tthrd = jnp.where(has_pos, thpsim - margin, 1.0 - margin)
    self_f = (tsims > tthrd).astype(jnp.float32) - tmask_f
    tcnt = jnp.sum(self_f, axis=1, keepdims=True)
    tsum = jnp.sum(self_f * _softplus_poly(tsims), axis=1, keepdims=True)
    thn = jnp.where(tcnt > 0.0, tsum / jnp.maximum(tcnt, 1.0), 0.0)
    thp = jnp.where(has_pos, _softplus(-thpsim), 0.0)
    th_ref[...] += thp + thn


def _finish_kernel(x_ref, pairs_ref, s_ref, tl_ref, th_ref, loss_ref,
                   *, C, P, margin, npad):
    B = x_ref.shape[0]
    x = x_ref[...]
    nsq_row = _rownorms_sq(x)
    norm_row = jnp.sqrt(jnp.maximum(nsq_row, 1e-24))

    bu = jnp.mean(jnp.log(s_ref[...] - float(npad)) - tl_ref[...] * norm_row)
    th_loss = jnp.sum(th_ref[...]) / C

    invn = jax.lax.rsqrt(
        jnp.maximum(jnp.sum(x * x, axis=1, keepdims=True), 1e-24))
    ninp = x * invn
    sim = jax.lax.dot_general(ninp, ninp, (((1,), (1,)), ((), ())),
                              preferred_element_type=jnp.float32)
    colid = jax.lax.broadcasted_iota(jnp.int32, (1, B), 1)
    pairs = pairs_ref[...]
    hp = jnp.full((B, 1), 2.0, jnp.float32)
    for q in range(P):
        pid = pairs[:, q:q + 1]
        ps = jnp.sum(jnp.where(colid == pid, sim, 0.0), axis=1, keepdims=True)
        hp = jnp.minimum(hp, ps)
    thr = hp - margin
    cnt = jnp.zeros((B, 1), jnp.float32)
    nsum = jnp.zeros((B, 1), jnp.float32)
    for q in range(P):
        nid = pairs[:, P + q:P + q + 1]
        ns = jnp.sum(jnp.where(colid == nid, sim, 0.0), axis=1, keepdims=True)
        sel = ns > thr
        cnt += sel.astype(jnp.float32)
        nsum += jnp.where(sel, _softplus(ns), 0.0)
    hn = jnp.where(cnt > 0.0, nsum / jnp.maximum(cnt, 1.0), 0.0)
    h_loss = jnp.mean(_softplus(-hp) + hn)

    loss_ref[...] = jnp.full(loss_ref.shape, bu + h_loss + th_loss,
                             jnp.float32)


def _run(inputs, V, tgt_row, pairs, tile_c, interpret=False):
    B, D = inputs.shape
    C = V.shape[0]
    P = pairs.shape[1] // 2
    grid = pl.cdiv(C, tile_c)
    npad = grid * tile_c - C

    main = functools.partial(_main_kernel, C=C, margin=_MARGIN)
    outT, s, tl, th = pl.pallas_call(
        main,
        grid=(grid,),
        in_specs=[
            pl.BlockSpec((B, D), lambda j: (0, 0)),
            pl.BlockSpec((tile_c, D), lambda j: (j, 0)),
            pl.BlockSpec((1, B), lambda j: (0, 0)),
        ],
        out_specs=[
            pl.BlockSpec((tile_c, B), lambda j: (j, 0)),
            pl.BlockSpec((1, B), lambda j: (0, 0)),
            pl.BlockSpec((1, B), lambda j: (0, 0)),
            pl.BlockSpec((tile_c, 1), lambda j: (0, 0)),
        ],
        out_shape=[
            jax.ShapeDtypeStruct((C, B), jnp.float32),
            jax.ShapeDtypeStruct((1, B), jnp.float32),
            jax.ShapeDtypeStruct((1, B), jnp.float32),
            jax.ShapeDtypeStruct((tile_c, 1), jnp.float32),
        ],
        scratch_shapes=[
            pltpu.VMEM((1, B), jnp.float32),
        ],
        compiler_params=pltpu.CompilerParams(
            dimension_semantics=("arbitrary",)),
        interpret=interpret,
    )(inputs, V, tgt_row)

    finish = functools.partial(_finish_kernel, C=C, P=P, margin=_MARGIN,
                               npad=npad)
    loss = pl.pallas_call(
        finish,
        out_shape=jax.ShapeDtypeStruct((8, 128), jnp.float32),
        interpret=interpret,
    )(inputs, pairs, s, tl, th)
    return outT.T, loss


def kernel(inputs, V, targets, label_to_pairs, indexs):
    B, D = inputs.shape
    P = label_to_pairs.shape[2]
    tgt_row = targets.astype(jnp.int32).reshape(1, B)
    pairs = label_to_pairs.astype(jnp.int32).reshape(B, 2 * P)
    out, loss = _run(inputs, V, tgt_row, pairs, _TILE_C)
    return loss[0, 0], out


# speedup vs baseline: 1.9291x; 1.0123x over previous
import functools

import jax
import jax.numpy as jnp
from jax.experimental import pallas as pl
from jax.experimental.pallas import tpu as pltpu

_MARGIN = 0.3
_TILE_C = 1792

_SP_C0 = 0.6931560237943111
_SP_C1 = 0.12482909105071764
_SP_C2 = -0.00472949478678556


def _softplus_poly(x):
    u = x * x
    return ((_SP_C2 * u + _SP_C1) * u + _SP_C0) + 0.5 * x


def _softplus(x):
    return jnp.logaddexp(x, 0.0)


def _rownorms_sq(x):
    ones = jnp.ones((1, x.shape[1]), jnp.float32)
    return jax.lax.dot_general(ones, x * x, (((1,), (1,)), ((), ())),
                               preferred_element_type=jnp.float32)


def _main_kernel(x_ref, v_ref, tgt_ref,
                 out_ref, s_ref, tl_ref, th_ref, invn_ref,
                 *, C, margin):
    j = pl.program_id(0)
    Ct = v_ref.shape[0]

    x = x_ref[...]

    @pl.when(j == 0)
    def _init():
        s_ref[...] = jnp.zeros(s_ref.shape, jnp.float32)
        tl_ref[...] = jnp.zeros(tl_ref.shape, jnp.float32)
        th_ref[...] = jnp.zeros(th_ref.shape, jnp.float32)
        invn_ref[...] = jax.lax.rsqrt(
            jnp.maximum(_rownorms_sq(x), 1e-24))

    blockT = jax.lax.dot_general(v_ref[...], x, (((1,), (1,)), ((), ())),
                                 preferred_element_type=jnp.float32)
    out_ref[...] = blockT

    rows = j * Ct + jax.lax.broadcasted_iota(jnp.int32, (Ct, 1), 0)
    tmask = rows == tgt_ref[...]
    blk = jnp.where(rows < C, blockT, 0.0)

    tmask_f = tmask.astype(jnp.float32)
    s_ref[...] += jnp.sum(jnp.exp(blk), axis=0, keepdims=True)
    invn = invn_ref[...]
    tsims = blk * invn
    tl_ref[...] += jnp.sum(tmask_f * tsims, axis=0, keepdims=True)
    thpsim_raw = jnp.min(jnp.where(tmask, tsims, 1e30), axis=1, keepdims=True)
    has_pos = thpsim_raw < 1e29
    thpsim = jnp.where(has_pos, thpsim_raw, 0.0)
    tthrd = jnp.where(has_pos, thpsim - margin, 1.0 - margin)
    self_f = (tsims > tthrd).astype(jnp.float32) - tmask_f
    tcnt = jnp.sum(self_f, axis=1, keepdims=True)
    tsum = jnp.sum(self_f * _softplus_poly(tsims), axis=1, keepdims=True)
    thn = jnp.where(tcnt > 0.0, tsum / jnp.maximum(tcnt, 1.0), 0.0)
    thp = jnp.where(has_pos, _softplus(-thpsim), 0.0)
    th_ref[...] += thp + thn


def _finish_kernel(x_ref, pairs_ref, s_ref, tl_ref, th_ref, loss_ref,
                   *, C, P, margin, npad):
    B = x_ref.shape[0]
    x = x_ref[...]
    nsq_row = _rownorms_sq(x)
    norm_row = jnp.sqrt(jnp.maximum(nsq_row, 1e-24))

    bu = jnp.mean(jnp.log(s_ref[...] - float(npad)) - tl_ref[...] * norm_row)
    th_loss = jnp.sum(th_ref[...]) / C

    invn = jax.lax.rsqrt(
        jnp.maximum(jnp.sum(x * x, axis=1, keepdims=True), 1e-24))
    ninp = x * invn
    sim = jax.lax.dot_general(ninp, ninp, (((1,), (1,)), ((), ())),
                              preferred_element_type=jnp.float32)
    colid = jax.lax.broadcasted_iota(jnp.int32, (1, B), 1)
    pairs = pairs_ref[...]
    hp = jnp.full((B, 1), 2.0, jnp.float32)
    for q in range(P):
        pid = pairs[:, q:q + 1]
        ps = jnp.sum(jnp.where(colid == pid, sim, 0.0), axis=1, keepdims=True)
        hp = jnp.minimum(hp, ps)
    thr = hp - margin
    cnt = jnp.zeros((B, 1), jnp.float32)
    nsum = jnp.zeros((B, 1), jnp.float32)
    for q in range(P):
        nid = pairs[:, P + q:P + q + 1]
        ns = jnp.sum(jnp.where(colid == nid, sim, 0.0), axis=1, keepdims=True)
        sel = ns > thr
        cnt += sel.astype(jnp.float32)
        nsum += jnp.where(sel, _softplus(ns), 0.0)
    hn = jnp.where(cnt > 0.0, nsum / jnp.maximum(cnt, 1.0), 0.0)
    h_loss = jnp.mean(_softplus(-hp) + hn)

    loss_ref[...] = jnp.full(loss_ref.shape, bu + h_loss + th_loss,
                             jnp.float32)


def _run(inputs, V, tgt_row, pairs, tile_c, interpret=False):
    B, D = inputs.shape
    C = V.shape[0]
    P = pairs.shape[1] // 2
    grid = pl.cdiv(C, tile_c)
    npad = grid * tile_c - C

    main = functools.partial(_main_kernel, C=C, margin=_MARGIN)
    outT, s, tl, th = pl.pallas_call(
        main,
        grid=(grid,),
        in_specs=[
            pl.BlockSpec((B, D), lambda j: (0, 0)),
            pl.BlockSpec((tile_c, D), lambda j: (j, 0)),
            pl.BlockSpec((1, B), lambda j: (0, 0)),
        ],
        out_specs=[
            pl.BlockSpec((tile_c, B), lambda j: (j, 0)),
            pl.BlockSpec((1, B), lambda j: (0, 0)),
            pl.BlockSpec((1, B), lambda j: (0, 0)),
            pl.BlockSpec((tile_c, 1), lambda j: (0, 0)),
        ],
        out_shape=[
            jax.ShapeDtypeStruct((C, B), jnp.float32),
            jax.ShapeDtypeStruct((1, B), jnp.float32),
            jax.ShapeDtypeStruct((1, B), jnp.float32),
            jax.ShapeDtypeStruct((tile_c, 1), jnp.float32),
        ],
        scratch_shapes=[
            pltpu.VMEM((1, B), jnp.float32),
        ],
        compiler_params=pltpu.CompilerParams(
            dimension_semantics=("arbitrary",)),
        interpret=interpret,
    )(inputs, V, tgt_row)

    finish = functools.partial(_finish_kernel, C=C, P=P, margin=_MARGIN,
                               npad=npad)
    loss = pl.pallas_call(
        finish,
        out_shape=jax.ShapeDtypeStruct((8, 128), jnp.float32),
        interpret=interpret,
    )(inputs, pairs, s, tl, th)
    return outT.T, loss


def kernel(inputs, V, targets, label_to_pairs, indexs):
    B, D = inputs.shape
    P = label_to_pairs.shape[2]
    tgt_row = targets.astype(jnp.int32).reshape(1, B)
    pairs = label_to_pairs.astype(jnp.int32).reshape(B, 2 * P)
    out, loss = _run(inputs, V, tgt_row, pairs, _TILE_C)
    return loss[0, 0], out
